# baseline (speedup 1.0000x reference)
import functools

import jax
import jax.numpy as jnp
from jax import lax
from jax.experimental import pallas as pl
from jax.experimental.pallas import tpu as pltpu
from jax.experimental.pallas import tpu_sc as plsc

NUM_FRAME = 1000000
BATCH = 16384
NC = 2
NS = 16
L = 16
NW = NC * NS
BPW = BATCH // NW
STEPS = BPW // L

_ANGLE_SCALE = 0.2 / 180.0 * 3.14159265358979323846


def _pose_body(idx_hbm, rot_hbm, tr_hbm, out_hbm, idx_v, idx3_v, rows_r, rows_t, out_v, sem):
    wid = lax.axis_index("s") * NC + lax.axis_index("c")
    base = wid * BPW

    pltpu.sync_copy(idx_hbm.at[pl.ds(base, BPW)], idx_v)

    def expand(i, carry):
        v = idx_v[pl.ds(i * L, L)]
        v3 = v * 3
        idx3_v[pl.ds(i * L, L)] = v3
        idx3_v[pl.ds(BPW + i * L, L)] = v3 + 1
        idx3_v[pl.ds(2 * BPW + i * L, L)] = v3 + 2
        return carry

    lax.fori_loop(0, STEPS, expand, 0)

    cp_r = pltpu.async_copy(rot_hbm.at[idx3_v], rows_r, sem)
    cp_t = pltpu.async_copy(tr_hbm.at[idx3_v], rows_t, sem)
    cp_r.wait()
    cp_t.wait()

    lane = lax.iota(jnp.int32, L)
    zeros = jnp.zeros((L,), jnp.float32)
    ones = jnp.ones((L,), jnp.float32)

    def tanh(x):
        return 1.0 - 2.0 / (jnp.exp(2.0 * x) + 1.0)

    def step(i, carry):
        o = i * L
        rx = rows_r[pl.ds(o, L)]
        ry = rows_r[pl.ds(BPW + o, L)]
        rz = rows_r[pl.ds(2 * BPW + o, L)]
        tx = rows_t[pl.ds(o, L)]
        ty = rows_t[pl.ds(BPW + o, L)]
        tz = rows_t[pl.ds(2 * BPW + o, L)]

        ax = _ANGLE_SCALE * tanh(rx)
        ay = _ANGLE_SCALE * tanh(ry)
        az = _ANGLE_SCALE * tanh(rz)
        t2 = ax * ax + ay * ay + az * az
        h2 = 0.25 * t2
        cos_h = 1.0 - 0.5 * h2 + (1.0 / 24.0) * h2 * h2
        s = 0.5 - (1.0 / 48.0) * t2 + (1.0 / 3840.0) * t2 * t2
        qr = cos_h
        qi = ax * s
        qj = ay * s
        qk = az * s
        two_s = 2.0 / (qr * qr + qi * qi + qj * qj + qk * qk)

        m00 = 1.0 - two_s * (qj * qj + qk * qk)
        m01 = two_s * (qi * qj - qk * qr)
        m02 = two_s * (qi * qk + qj * qr)
        m10 = two_s * (qi * qj + qk * qr)
        m11 = 1.0 - two_s * (qi * qi + qk * qk)
        m12 = two_s * (qj * qk - qi * qr)
        m20 = two_s * (qi * qk - qj * qr)
        m21 = two_s * (qj * qk + qi * qr)
        m22 = 1.0 - two_s * (qi * qi + qj * qj)
        t0 = 0.05 * tanh(tx)
        t1 = 0.05 * tanh(ty)
        t_2 = 0.05 * tanh(tz)

        vals = (m00, m01, m02, t0, m10, m11, m12, t1,
                m20, m21, m22, t_2, zeros, zeros, zeros, ones)
        for c, v in enumerate(vals):
            out_v[c, pl.ds(o, L)] = v
        return carry

    lax.fori_loop(0, STEPS, step, 0)
    pltpu.sync_copy(out_v, out_hbm.at[:, pl.ds(base, BPW)])


@functools.partial(
    pl.kernel,
    out_type=jax.ShapeDtypeStruct((16, BATCH), jnp.float32),
    mesh=plsc.VectorSubcoreMesh(core_axis_name="c", subcore_axis_name="s"),
    scratch_types=[
        pltpu.VMEM((BPW,), jnp.int32),
        pltpu.VMEM((3 * BPW,), jnp.int32),
        pltpu.VMEM((3 * BPW,), jnp.float32),
        pltpu.VMEM((3 * BPW,), jnp.float32),
        pltpu.VMEM((16, BPW), jnp.float32),
        pltpu.SemaphoreType.DMA,
    ],
)
def _pose_kernel(idx_hbm, rot_hbm, tr_hbm, out_hbm, idx_v, idx3_v, rows_r, rows_t, out_v, sem):
    _pose_body(idx_hbm, rot_hbm, tr_hbm, out_hbm, idx_v, idx3_v, rows_r, rows_t, out_v, sem)


def kernel(frame_idx, camera_idx, rotations, translations):
    del camera_idx
    idx = frame_idx.astype(jnp.int32)
    cols = _pose_kernel(idx, rotations.reshape(-1), translations.reshape(-1))
    return cols.T.reshape(BATCH, 4, 4)


# speedup vs baseline: 65.2187x; 65.2187x over previous
import functools

import jax
import jax.numpy as jnp
from jax import lax
from jax.experimental import pallas as pl
from jax.experimental.pallas import tpu as pltpu
from jax.experimental.pallas import tpu_sc as plsc

NUM_FRAME = 1000000
BATCH = 16384
NC = 2
NS = 16
L = 16
NW = NC * NS
BPW = BATCH // NW
STEPS = BPW // L

_ANGLE_SCALE = 0.2 / 180.0 * 3.14159265358979323846


def _pose_body(idx_hbm, rot_hbm, tr_hbm, out_hbm, idx_v, idx3_v, rows_r, rows_t, out_v, sem):
    wid = lax.axis_index("s") * NC + lax.axis_index("c")
    base = wid * BPW

    pltpu.sync_copy(idx_hbm.at[pl.ds(base, BPW)], idx_v)

    def expand(i, carry):
        v = idx_v[pl.ds(i * L, L)]
        idx3_v[pl.ds(i * L, L)] = v
        idx3_v[pl.ds(BPW + i * L, L)] = v + NUM_FRAME
        idx3_v[pl.ds(2 * BPW + i * L, L)] = v + 2 * NUM_FRAME
        return carry

    lax.fori_loop(0, STEPS, expand, 0)

    cp_r = pltpu.async_copy(rot_hbm.at[idx3_v], rows_r, sem)
    cp_t = pltpu.async_copy(tr_hbm.at[idx3_v], rows_t, sem)
    cp_r.wait()
    cp_t.wait()

    zeros = jnp.zeros((L,), jnp.float32)
    ones = jnp.ones((L,), jnp.float32)

    def tanh(x):
        return 1.0 - 2.0 / (jnp.exp(2.0 * x) + 1.0)

    def step(i, carry):
        o = i * L
        rx = rows_r[pl.ds(o, L)]
        ry = rows_r[pl.ds(BPW + o, L)]
        rz = rows_r[pl.ds(2 * BPW + o, L)]
        tx = rows_t[pl.ds(o, L)]
        ty = rows_t[pl.ds(BPW + o, L)]
        tz = rows_t[pl.ds(2 * BPW + o, L)]

        ax = _ANGLE_SCALE * tanh(rx)
        ay = _ANGLE_SCALE * tanh(ry)
        az = _ANGLE_SCALE * tanh(rz)
        t2 = ax * ax + ay * ay + az * az
        h2 = 0.25 * t2
        cos_h = 1.0 - 0.5 * h2 + (1.0 / 24.0) * h2 * h2
        s = 0.5 - (1.0 / 48.0) * t2 + (1.0 / 3840.0) * t2 * t2
        qr = cos_h
        qi = ax * s
        qj = ay * s
        qk = az * s
        two_s = 2.0 / (qr * qr + qi * qi + qj * qj + qk * qk)

        m00 = 1.0 - two_s * (qj * qj + qk * qk)
        m01 = two_s * (qi * qj - qk * qr)
        m02 = two_s * (qi * qk + qj * qr)
        m10 = two_s * (qi * qj + qk * qr)
        m11 = 1.0 - two_s * (qi * qi + qk * qk)
        m12 = two_s * (qj * qk - qi * qr)
        m20 = two_s * (qi * qk - qj * qr)
        m21 = two_s * (qj * qk + qi * qr)
        m22 = 1.0 - two_s * (qi * qi + qj * qj)
        t0 = 0.05 * tanh(tx)
        t1 = 0.05 * tanh(ty)
        t_2 = 0.05 * tanh(tz)

        vals = (m00, m01, m02, t0, m10, m11, m12, t1,
                m20, m21, m22, t_2, zeros, zeros, zeros, ones)
        for c, v in enumerate(vals):
            out_v[c, pl.ds(o, L)] = v
        return carry

    lax.fori_loop(0, STEPS, step, 0)
    pltpu.sync_copy(out_v, out_hbm.at[:, pl.ds(base, BPW)])


@functools.partial(
    pl.kernel,
    out_type=jax.ShapeDtypeStruct((16, BATCH), jnp.float32),
    mesh=plsc.VectorSubcoreMesh(core_axis_name="c", subcore_axis_name="s"),
    compiler_params=pltpu.CompilerParams(use_tc_tiling_on_sc=False),
    scratch_types=[
        pltpu.VMEM((BPW,), jnp.int32),
        pltpu.VMEM((3 * BPW,), jnp.int32),
        pltpu.VMEM((3 * BPW,), jnp.float32),
        pltpu.VMEM((3 * BPW,), jnp.float32),
        pltpu.VMEM((16, BPW), jnp.float32),
        pltpu.SemaphoreType.DMA,
    ],
)
def _pose_kernel(idx_hbm, rot_hbm, tr_hbm, out_hbm, idx_v, idx3_v, rows_r, rows_t, out_v, sem):
    _pose_body(idx_hbm, rot_hbm, tr_hbm, out_hbm, idx_v, idx3_v, rows_r, rows_t, out_v, sem)


def kernel(frame_idx, camera_idx, rotations, translations):
    del camera_idx
    idx = frame_idx.astype(jnp.int32)
    rot_flat = rotations.T.reshape(3 * NUM_FRAME)
    tr_flat = translations.T.reshape(3 * NUM_FRAME)
    cols = _pose_kernel(idx, rot_flat, tr_flat)
    return cols.T.reshape(BATCH, 4, 4)
